# baseline (device time: 37214 ns/iter reference)
import jax
import jax.numpy as jnp
from jax import lax
from jax.experimental import pallas as pl
from jax.experimental.pallas import tpu as pltpu

N_DEV = 16
B, SQ, SKV, HQ, DH = 2, 256, 256, 64, 64
H_LOC = HQ // N_DEV
D_MODEL = 512
ROWS = B * SQ
CHUNK = ROWS // N_DEV
BLK = 64


def _body(x_ref, wq_ref, k_ref, v_ref, wo_ref, out_ref,
          partial_ref, rs_ref, gat_ref,
          rs_send, rs_recv, ag_send, ag_recv):
    me = lax.axis_index("i")

    barrier = pltpu.get_barrier_semaphore()
    for off in range(1, N_DEV):
        peer = lax.rem(me + off, N_DEV)
        pl.semaphore_signal(
            barrier, inc=1,
            device_id=(peer,), device_id_type=pl.DeviceIdType.MESH,
        )
    pl.semaphore_wait(barrier, N_DEV - 1)

    wq = wq_ref[:]
    wo = wo_ref[:]
    for b in range(B):
        q_all = jnp.dot(x_ref[b], wq,
                        preferred_element_type=jnp.float32)
        ctxs = []
        for h in range(H_LOC):
            q = q_all[:, h * DH:(h + 1) * DH].astype(jnp.bfloat16)
            k = k_ref[b, h]
            s = lax.dot_general(
                q, k, (((1,), (1,)), ((), ())),
                preferred_element_type=jnp.float32) * 0.125
            row_blk = lax.broadcasted_iota(jnp.int32, (SQ, SKV), 0) // BLK
            col_blk = lax.broadcasted_iota(jnp.int32, (SQ, SKV), 1) // BLK
            s = jnp.where(col_blk <= row_blk, s, -1e9)
            m = jnp.max(s, axis=1, keepdims=True)
            w = jnp.exp(s - m)
            w = w / jnp.sum(w, axis=1, keepdims=True)
            ctxs.append(jnp.dot(w.astype(jnp.bfloat16), v_ref[b, h],
                                preferred_element_type=jnp.float32))
        ctx = jnp.concatenate(ctxs, axis=1).astype(jnp.bfloat16)
        partial_ref[b * SQ:(b + 1) * SQ, :] = jnp.dot(
            ctx, wo, preferred_element_type=jnp.float32)

    rs_sends = []
    for off in range(1, N_DEV):
        dest = lax.rem(me + off, N_DEV)
        rdma = pltpu.make_async_remote_copy(
            src_ref=partial_ref.at[pl.ds(dest * CHUNK, CHUNK), :],
            dst_ref=rs_ref.at[pl.ds(me * CHUNK, CHUNK), :],
            send_sem=rs_send.at[dest],
            recv_sem=rs_recv.at[me],
            device_id=(dest,),
            device_id_type=pl.DeviceIdType.MESH,
        )
        rdma.start()
        rs_sends.append(rdma)
    rs_ref[pl.ds(me * CHUNK, CHUNK), :] = partial_ref[pl.ds(me * CHUNK, CHUNK), :]
    for off in range(1, N_DEV):
        src = lax.rem(me + off, N_DEV)
        recv = pltpu.make_async_remote_copy(
            src_ref=partial_ref.at[pl.ds(0, CHUNK), :],
            dst_ref=rs_ref.at[pl.ds(src * CHUNK, CHUNK), :],
            send_sem=rs_send.at[src],
            recv_sem=rs_recv.at[src],
            device_id=(src,),
            device_id_type=pl.DeviceIdType.MESH,
        )
        recv.wait_recv()
    for rdma in rs_sends:
        rdma.wait_send()

    total = rs_ref[0:CHUNK, :]
    for s in range(1, N_DEV):
        total = total + rs_ref[s * CHUNK:(s + 1) * CHUNK, :]
    gat_ref[pl.ds(me * CHUNK, CHUNK), :] = total

    ag_sends = []
    for off in range(1, N_DEV):
        dest = lax.rem(me + off, N_DEV)
        rdma = pltpu.make_async_remote_copy(
            src_ref=gat_ref.at[pl.ds(me * CHUNK, CHUNK), :],
            dst_ref=gat_ref.at[pl.ds(me * CHUNK, CHUNK), :],
            send_sem=ag_send.at[dest],
            recv_sem=ag_recv.at[me],
            device_id=(dest,),
            device_id_type=pl.DeviceIdType.MESH,
        )
        rdma.start()
        ag_sends.append(rdma)
    for off in range(1, N_DEV):
        src = lax.rem(me + off, N_DEV)
        recv = pltpu.make_async_remote_copy(
            src_ref=partial_ref.at[pl.ds(0, CHUNK), :],
            dst_ref=gat_ref.at[pl.ds(src * CHUNK, CHUNK), :],
            send_sem=ag_send.at[src],
            recv_sem=ag_recv.at[src],
            device_id=(src,),
            device_id_type=pl.DeviceIdType.MESH,
        )
        recv.wait_recv()
    for rdma in ag_sends:
        rdma.wait_send()

    for b in range(B):
        out_ref[b, :, :] = gat_ref[b * SQ:(b + 1) * SQ, :]


def kernel(x, Wq, K_ext, V_ext, Wo):
    p = lax.axis_index("i")
    Ks = lax.dynamic_slice_in_dim(K_ext, p * H_LOC, H_LOC, axis=2)
    Vs = lax.dynamic_slice_in_dim(V_ext, p * H_LOC, H_LOC, axis=2)
    Ks = jnp.transpose(Ks, (0, 2, 1, 3)).astype(jnp.bfloat16)
    Vs = jnp.transpose(Vs, (0, 2, 1, 3)).astype(jnp.bfloat16)

    return pl.pallas_call(
        _body,
        out_shape=jax.ShapeDtypeStruct((B, SQ, D_MODEL), jnp.float32),
        in_specs=[pl.BlockSpec(memory_space=pltpu.VMEM)] * 5,
        out_specs=pl.BlockSpec(memory_space=pltpu.VMEM),
        scratch_shapes=[
            pltpu.VMEM((ROWS, D_MODEL), jnp.float32),
            pltpu.VMEM((ROWS, D_MODEL), jnp.float32),
            pltpu.VMEM((ROWS, D_MODEL), jnp.float32),
            pltpu.SemaphoreType.DMA((N_DEV,)),
            pltpu.SemaphoreType.DMA((N_DEV,)),
            pltpu.SemaphoreType.DMA((N_DEV,)),
            pltpu.SemaphoreType.DMA((N_DEV,)),
        ],
        compiler_params=pltpu.CompilerParams(collective_id=0),
    )(x.astype(jnp.bfloat16), Wq.astype(jnp.bfloat16), Ks, Vs,
      Wo.astype(jnp.bfloat16))


# device time: 28516 ns/iter; 1.3050x vs baseline; 1.3050x over previous
import jax
import jax.numpy as jnp
from jax import lax
from jax.experimental import pallas as pl
from jax.experimental.pallas import tpu as pltpu

N_DEV = 16
B, SQ, SKV, HQ, DH = 2, 256, 256, 64, 64
H_LOC = HQ // N_DEV
D_MODEL = 512
ROWS = B * SQ
CHUNK = ROWS // N_DEV
BLK = 64


def _body(x_ref, wq_ref, k_ref, v_ref, wo_ref, out_ref,
          partial_ref, rs_ref, gat_ref,
          rs_send, rs_recv, ag_send, ag_recv):
    me = lax.axis_index("i")

    barrier = pltpu.get_barrier_semaphore()
    for off in range(1, N_DEV):
        peer = lax.rem(me + off, N_DEV)
        pl.semaphore_signal(
            barrier, inc=1,
            device_id=(peer,), device_id_type=pl.DeviceIdType.MESH,
        )
    pl.semaphore_wait(barrier, N_DEV - 1)

    wq = wq_ref[:]
    wo = wo_ref[:]
    for b in range(B):
        q_all = jnp.dot(x_ref[b], wq,
                        preferred_element_type=jnp.float32)
        ctxs = []
        for h in range(H_LOC):
            q = q_all[:, h * DH:(h + 1) * DH].astype(jnp.bfloat16)
            k = k_ref[b, h]
            s = lax.dot_general(
                q, k, (((1,), (1,)), ((), ())),
                preferred_element_type=jnp.float32) * 0.125
            row_blk = lax.broadcasted_iota(jnp.int32, (SQ, SKV), 0) // BLK
            col_blk = lax.broadcasted_iota(jnp.int32, (SQ, SKV), 1) // BLK
            s = jnp.where(col_blk <= row_blk, s, -1e9)
            m = jnp.max(s, axis=1, keepdims=True)
            w = jnp.exp(s - m)
            w = w / jnp.sum(w, axis=1, keepdims=True)
            ctxs.append(jnp.dot(w.astype(jnp.bfloat16), v_ref[b, h],
                                preferred_element_type=jnp.float32))
        ctx = jnp.concatenate(ctxs, axis=1).astype(jnp.bfloat16)
        partial_ref[b * SQ:(b + 1) * SQ, :] = jnp.dot(
            ctx, wo, preferred_element_type=jnp.float32
        ).astype(jnp.bfloat16)

    rs_sends = []
    for off in range(1, N_DEV):
        dest = lax.rem(me + off, N_DEV)
        rdma = pltpu.make_async_remote_copy(
            src_ref=partial_ref.at[pl.ds(dest * CHUNK, CHUNK), :],
            dst_ref=rs_ref.at[pl.ds(me * CHUNK, CHUNK), :],
            send_sem=rs_send.at[dest],
            recv_sem=rs_recv.at[me],
            device_id=(dest,),
            device_id_type=pl.DeviceIdType.MESH,
        )
        rdma.start()
        rs_sends.append(rdma)
    rs_ref[pl.ds(me * CHUNK, CHUNK), :] = partial_ref[pl.ds(me * CHUNK, CHUNK), :]
    for off in range(1, N_DEV):
        src = lax.rem(me + off, N_DEV)
        recv = pltpu.make_async_remote_copy(
            src_ref=partial_ref.at[pl.ds(0, CHUNK), :],
            dst_ref=rs_ref.at[pl.ds(src * CHUNK, CHUNK), :],
            send_sem=rs_send.at[src],
            recv_sem=rs_recv.at[src],
            device_id=(src,),
            device_id_type=pl.DeviceIdType.MESH,
        )
        recv.wait_recv()
    for rdma in rs_sends:
        rdma.wait_send()

    total = rs_ref[0:CHUNK, :].astype(jnp.float32)
    for s in range(1, N_DEV):
        total = total + rs_ref[s * CHUNK:(s + 1) * CHUNK, :].astype(jnp.float32)
    gat_ref[pl.ds(me * CHUNK, CHUNK), :] = total.astype(jnp.bfloat16)

    ag_sends = []
    for off in range(1, N_DEV):
        dest = lax.rem(me + off, N_DEV)
        rdma = pltpu.make_async_remote_copy(
            src_ref=gat_ref.at[pl.ds(me * CHUNK, CHUNK), :],
            dst_ref=gat_ref.at[pl.ds(me * CHUNK, CHUNK), :],
            send_sem=ag_send.at[dest],
            recv_sem=ag_recv.at[me],
            device_id=(dest,),
            device_id_type=pl.DeviceIdType.MESH,
        )
        rdma.start()
        ag_sends.append(rdma)
    for off in range(1, N_DEV):
        src = lax.rem(me + off, N_DEV)
        recv = pltpu.make_async_remote_copy(
            src_ref=partial_ref.at[pl.ds(0, CHUNK), :],
            dst_ref=gat_ref.at[pl.ds(src * CHUNK, CHUNK), :],
            send_sem=ag_send.at[src],
            recv_sem=ag_recv.at[src],
            device_id=(src,),
            device_id_type=pl.DeviceIdType.MESH,
        )
        recv.wait_recv()
    for rdma in ag_sends:
        rdma.wait_send()

    for b in range(B):
        out_ref[b, :, :] = gat_ref[b * SQ:(b + 1) * SQ, :].astype(jnp.float32)


def kernel(x, Wq, K_ext, V_ext, Wo):
    p = lax.axis_index("i")
    Ks = lax.dynamic_slice_in_dim(K_ext, p * H_LOC, H_LOC, axis=2)
    Vs = lax.dynamic_slice_in_dim(V_ext, p * H_LOC, H_LOC, axis=2)
    Ks = jnp.transpose(Ks, (0, 2, 1, 3)).astype(jnp.bfloat16)
    Vs = jnp.transpose(Vs, (0, 2, 1, 3)).astype(jnp.bfloat16)

    return pl.pallas_call(
        _body,
        out_shape=jax.ShapeDtypeStruct((B, SQ, D_MODEL), jnp.float32),
        in_specs=[pl.BlockSpec(memory_space=pltpu.VMEM)] * 5,
        out_specs=pl.BlockSpec(memory_space=pltpu.VMEM),
        scratch_shapes=[
            pltpu.VMEM((ROWS, D_MODEL), jnp.bfloat16),
            pltpu.VMEM((ROWS, D_MODEL), jnp.bfloat16),
            pltpu.VMEM((ROWS, D_MODEL), jnp.bfloat16),
            pltpu.SemaphoreType.DMA((N_DEV,)),
            pltpu.SemaphoreType.DMA((N_DEV,)),
            pltpu.SemaphoreType.DMA((N_DEV,)),
            pltpu.SemaphoreType.DMA((N_DEV,)),
        ],
        compiler_params=pltpu.CompilerParams(collective_id=0),
    )(x.astype(jnp.bfloat16), Wq.astype(jnp.bfloat16), Ks, Vs,
      Wo.astype(jnp.bfloat16))


# device time: 27105 ns/iter; 1.3730x vs baseline; 1.0521x over previous
import jax
import jax.numpy as jnp
from jax import lax
from jax.experimental import pallas as pl
from jax.experimental.pallas import tpu as pltpu

N_DEV = 16
B, SQ, SKV, HQ, DH = 2, 256, 256, 64, 64
H_LOC = HQ // N_DEV
D_MODEL = 512
ROWS = B * SQ
CHUNK = ROWS // N_DEV
BLK = 64


def _body(x_ref, wq_ref, k_ref, v_ref, wo_ref, out_ref,
          partial_ref, rs_ref, gat_ref,
          rs_send, rs_recv, ag_send, ag_recv):
    me = lax.axis_index("i")

    barrier = pltpu.get_barrier_semaphore()
    for off in range(1, N_DEV):
        peer = lax.rem(me + off, N_DEV)
        pl.semaphore_signal(
            barrier, inc=1,
            device_id=(peer,), device_id_type=pl.DeviceIdType.MESH,
        )

    wq = wq_ref[:]
    wo = wo_ref[:]
    rs_sends = []
    chunks_per_b = SQ // CHUNK
    for b in range(B):
        q_all = jnp.dot(x_ref[b], wq,
                        preferred_element_type=jnp.float32)
        ctxs = []
        for h in range(H_LOC):
            q = q_all[:, h * DH:(h + 1) * DH].astype(jnp.bfloat16)
            k = k_ref[b, h]
            s = lax.dot_general(
                q, k, (((1,), (1,)), ((), ())),
                preferred_element_type=jnp.float32) * 0.125
            row_blk = lax.broadcasted_iota(jnp.int32, (SQ, SKV), 0) // BLK
            col_blk = lax.broadcasted_iota(jnp.int32, (SQ, SKV), 1) // BLK
            s = jnp.where(col_blk <= row_blk, s, -1e9)
            m = jnp.max(s, axis=1, keepdims=True)
            w = jnp.exp(s - m)
            w = w / jnp.sum(w, axis=1, keepdims=True)
            ctxs.append(jnp.dot(w.astype(jnp.bfloat16), v_ref[b, h],
                                preferred_element_type=jnp.float32))
        ctx = jnp.concatenate(ctxs, axis=1).astype(jnp.bfloat16)
        partial_ref[b * SQ:(b + 1) * SQ, :] = jnp.dot(
            ctx, wo, preferred_element_type=jnp.float32
        ).astype(jnp.bfloat16)

        if b == 0:
            pl.semaphore_wait(barrier, N_DEV - 1)
        for dest in range(b * chunks_per_b, (b + 1) * chunks_per_b):
            rdma = pltpu.make_async_remote_copy(
                src_ref=partial_ref.at[pl.ds(dest * CHUNK, CHUNK), :],
                dst_ref=rs_ref.at[pl.ds(me * CHUNK, CHUNK), :],
                send_sem=rs_send.at[dest],
                recv_sem=rs_recv.at[me],
                device_id=(dest,),
                device_id_type=pl.DeviceIdType.MESH,
            )

            @pl.when(dest != me)
            def _(rdma=rdma):
                rdma.start()

            rs_sends.append((dest, rdma))

    rs_ref[pl.ds(me * CHUNK, CHUNK), :] = partial_ref[pl.ds(me * CHUNK, CHUNK), :]
    for off in range(1, N_DEV):
        src = lax.rem(me + off, N_DEV)
        recv = pltpu.make_async_remote_copy(
            src_ref=partial_ref.at[pl.ds(0, CHUNK), :],
            dst_ref=rs_ref.at[pl.ds(src * CHUNK, CHUNK), :],
            send_sem=rs_send.at[src],
            recv_sem=rs_recv.at[src],
            device_id=(src,),
            device_id_type=pl.DeviceIdType.MESH,
        )
        recv.wait_recv()
    for dest, rdma in rs_sends:
        @pl.when(dest != me)
        def _(rdma=rdma):
            rdma.wait_send()

    total = rs_ref[0:CHUNK, :].astype(jnp.float32)
    for s in range(1, N_DEV):
        total = total + rs_ref[s * CHUNK:(s + 1) * CHUNK, :].astype(jnp.float32)
    gat_ref[pl.ds(me * CHUNK, CHUNK), :] = total.astype(jnp.bfloat16)

    ag_sends = []
    for off in range(1, N_DEV):
        dest = lax.rem(me + off, N_DEV)
        rdma = pltpu.make_async_remote_copy(
            src_ref=gat_ref.at[pl.ds(me * CHUNK, CHUNK), :],
            dst_ref=gat_ref.at[pl.ds(me * CHUNK, CHUNK), :],
            send_sem=ag_send.at[dest],
            recv_sem=ag_recv.at[me],
            device_id=(dest,),
            device_id_type=pl.DeviceIdType.MESH,
        )
        rdma.start()
        ag_sends.append(rdma)
    for off in range(1, N_DEV):
        src = lax.rem(me + off, N_DEV)
        recv = pltpu.make_async_remote_copy(
            src_ref=partial_ref.at[pl.ds(0, CHUNK), :],
            dst_ref=gat_ref.at[pl.ds(src * CHUNK, CHUNK), :],
            send_sem=ag_send.at[src],
            recv_sem=ag_recv.at[src],
            device_id=(src,),
            device_id_type=pl.DeviceIdType.MESH,
        )
        recv.wait_recv()
    for rdma in ag_sends:
        rdma.wait_send()

    for b in range(B):
        out_ref[b, :, :] = gat_ref[b * SQ:(b + 1) * SQ, :].astype(jnp.float32)


def kernel(x, Wq, K_ext, V_ext, Wo):
    p = lax.axis_index("i")
    Ks = lax.dynamic_slice_in_dim(K_ext, p * H_LOC, H_LOC, axis=2)
    Vs = lax.dynamic_slice_in_dim(V_ext, p * H_LOC, H_LOC, axis=2)
    Ks = jnp.transpose(Ks, (0, 2, 1, 3)).astype(jnp.bfloat16)
    Vs = jnp.transpose(Vs, (0, 2, 1, 3)).astype(jnp.bfloat16)

    return pl.pallas_call(
        _body,
        out_shape=jax.ShapeDtypeStruct((B, SQ, D_MODEL), jnp.float32),
        in_specs=[pl.BlockSpec(memory_space=pltpu.VMEM)] * 5,
        out_specs=pl.BlockSpec(memory_space=pltpu.VMEM),
        scratch_shapes=[
            pltpu.VMEM((ROWS, D_MODEL), jnp.bfloat16),
            pltpu.VMEM((ROWS, D_MODEL), jnp.bfloat16),
            pltpu.VMEM((ROWS, D_MODEL), jnp.bfloat16),
            pltpu.SemaphoreType.DMA((N_DEV,)),
            pltpu.SemaphoreType.DMA((N_DEV,)),
            pltpu.SemaphoreType.DMA((N_DEV,)),
            pltpu.SemaphoreType.DMA((N_DEV,)),
        ],
        compiler_params=pltpu.CompilerParams(collective_id=0),
    )(x.astype(jnp.bfloat16), Wq.astype(jnp.bfloat16), Ks, Vs,
      Wo.astype(jnp.bfloat16))
